# baseline (speedup 1.0000x reference)
import functools
from math import sqrt, pi

import jax
import jax.numpy as jnp
from jax import lax
from jax.experimental import pallas as pl
from jax.experimental.pallas import tpu as pltpu
from jax.experimental.pallas import tpu_sc as plsc

NKERNEL = 128
POSINF = 10.0
EPS = 1e-05

_B = 16
_N = 128
_ROWS = _B * _N
_NW = 32
_RPW = _ROWS // _NW
_NA = 121
_NAPAD = 128

_F32_MIN = jnp.finfo(jnp.float32).min


def _sc_gather(atoms_flat, a_pad, b_pad):
    mesh = plsc.VectorSubcoreMesh(core_axis_name="c", subcore_axis_name="s")

    @functools.partial(
        pl.kernel,
        out_type=(
            jax.ShapeDtypeStruct((_ROWS, _N), jnp.float32),
            jax.ShapeDtypeStruct((_ROWS, _N), jnp.float32),
        ),
        mesh=mesh,
        compiler_params=pltpu.CompilerParams(needs_layout_passes=False),
        scratch_types=[
            pltpu.VMEM((_NA, _NA), jnp.float32),
            pltpu.VMEM((_NA, _NA), jnp.float32),
            pltpu.VMEM((_N,), jnp.int32),
            pltpu.VMEM((_RPW, _N), jnp.float32),
            pltpu.VMEM((_RPW, _N), jnp.float32),
            pltpu.SemaphoreType.DMA,
            pltpu.SemaphoreType.DMA,
            pltpu.SemaphoreType.DMA,
        ],
    )
    def k(atoms_hbm, a_hbm, b_hbm, ag_hbm, bg_hbm,
          a_v, b_v, at_v, ag_v, bg_v, sem_a, sem_b, sem_t):
        wid = lax.axis_index("s") * 2 + lax.axis_index("c")
        row0 = wid * _RPW
        batch = wid // (_N // _RPW)
        i0 = (wid % (_N // _RPW)) * _RPW

        cp_a = pltpu.async_copy(a_hbm, a_v, sem_a)
        cp_b = pltpu.async_copy(b_hbm, b_v, sem_b)
        cp_t = pltpu.async_copy(atoms_hbm.at[pl.ds(batch * _N, _N)], at_v,
                                sem_t)
        cp_t.wait()
        cp_a.wait()
        cp_b.wait()

        ajs = [at_v[pl.ds(jc * 16, 16)] for jc in range(_N // 16)]

        def row_body(r):
            i_splat = jnp.full((16,), i0, jnp.int32) + r
            ai = plsc.load_gather(at_v, [i_splat])
            for jc, aj in enumerate(ajs):
                ag_v[r, pl.ds(jc * 16, 16)] = plsc.load_gather(a_v, [ai, aj])
                bg_v[r, pl.ds(jc * 16, 16)] = plsc.load_gather(b_v, [ai, aj])

        plsc.parallel_loop(0, _RPW, 1, unroll=4)(row_body)
        out_a = pltpu.async_copy(ag_v, ag_hbm.at[pl.ds(row0, _RPW)], sem_a)
        out_b = pltpu.async_copy(bg_v, bg_hbm.at[pl.ds(row0, _RPW)], sem_b)
        out_a.wait()
        out_b.wait()

    return k(atoms_flat, a_pad, b_pad)


_LOG2E = 1.4426950408889634


def _tc_rbf(ag2, bg2, d2, mu2, sigma2):
    rows_per_block = 128
    grid = (_ROWS // rows_per_block,)

    def body(ag_ref, bg_ref, d_ref, mu_ref, sig_ref, o_ref, const_ref):
        @pl.when(pl.program_id(0) == 0)
        def _():
            sig = sig_ref[...]
            const_ref[0:1, :] = mu_ref[...]
            const_ref[1:2, :] = (-0.5 * _LOG2E) / (sig * sig)
            const_ref[2:3, :] = -jnp.log2((jnp.abs(sig) + EPS) * sqrt(2.0 * pi))

        mu = const_ref[0:1, :].reshape(1, 1, NKERNEL)
        s2 = const_ref[1:2, :].reshape(1, 1, NKERNEL)
        lc = const_ref[2:3, :].reshape(1, 1, NKERNEL)
        d = d_ref[...]
        d = jnp.where(jnp.isnan(d), jnp.float32(0.0), d)
        d = jnp.where(d == jnp.inf, jnp.float32(POSINF), d)
        d = jnp.where(d == -jnp.inf, _F32_MIN, d)
        x = ag_ref[...] * d + bg_ref[...]
        u = x[:, :, None] - mu
        o_ref[...] = jnp.exp2((u * u) * s2 + lc)

    return pl.pallas_call(
        body,
        grid=grid,
        in_specs=[
            pl.BlockSpec((rows_per_block, _N), lambda i: (i, 0)),
            pl.BlockSpec((rows_per_block, _N), lambda i: (i, 0)),
            pl.BlockSpec((rows_per_block, _N), lambda i: (i, 0)),
            pl.BlockSpec((1, NKERNEL), lambda i: (0, 0)),
            pl.BlockSpec((1, NKERNEL), lambda i: (0, 0)),
        ],
        out_specs=pl.BlockSpec((rows_per_block, _N, NKERNEL),
                               lambda i: (i, 0, 0)),
        out_shape=jax.ShapeDtypeStruct((_ROWS, _N, NKERNEL), jnp.float32),
        scratch_shapes=[pltpu.VMEM((8, NKERNEL), jnp.float32)],
    )(ag2, bg2, d2, mu2, sigma2)


@jax.jit
def kernel(atoms, distances, mu, sigma, a, b):
    atoms_flat = atoms.reshape(-1).astype(jnp.int32)
    d2 = distances.reshape(_ROWS, _N)
    ag2, bg2 = _sc_gather(atoms_flat, a, b)
    g = _tc_rbf(ag2, bg2, d2, mu.reshape(1, NKERNEL), sigma.reshape(1, NKERNEL))
    return g.reshape(_B, _N, _N, NKERNEL)


# speedup vs baseline: 1.0109x; 1.0109x over previous
import functools
from math import sqrt, pi

import jax
import jax.numpy as jnp
from jax import lax
from jax.experimental import pallas as pl
from jax.experimental.pallas import tpu as pltpu
from jax.experimental.pallas import tpu_sc as plsc

NKERNEL = 128
POSINF = 10.0
EPS = 1e-05

_B = 16
_N = 128
_ROWS = _B * _N
_NW = 32
_RPW = _ROWS // _NW
_NA = 121
_NAPAD = 128

_F32_MIN = jnp.finfo(jnp.float32).min


def _sc_gather(atoms_flat, a_pad, b_pad):
    mesh = plsc.VectorSubcoreMesh(core_axis_name="c", subcore_axis_name="s")

    @functools.partial(
        pl.kernel,
        out_type=(
            jax.ShapeDtypeStruct((_ROWS, _N), jnp.float32),
            jax.ShapeDtypeStruct((_ROWS, _N), jnp.float32),
        ),
        mesh=mesh,
        compiler_params=pltpu.CompilerParams(needs_layout_passes=False),
        scratch_types=[
            pltpu.VMEM((_NA, _NAPAD), jnp.float32),
            pltpu.VMEM((_NA, _NAPAD), jnp.float32),
            pltpu.VMEM((_N,), jnp.int32),
            pltpu.VMEM((_RPW, _N), jnp.float32),
            pltpu.VMEM((_RPW, _N), jnp.float32),
            pltpu.SemaphoreType.DMA,
            pltpu.SemaphoreType.DMA,
            pltpu.SemaphoreType.DMA,
        ],
    )
    def k(atoms_hbm, a_hbm, b_hbm, ag_hbm, bg_hbm,
          a_v, b_v, at_v, ag_v, bg_v, sem_a, sem_b, sem_t):
        wid = lax.axis_index("s") * 2 + lax.axis_index("c")
        row0 = wid * _RPW
        batch = wid // (_N // _RPW)
        i0 = (wid % (_N // _RPW)) * _RPW

        cp_a = pltpu.async_copy(a_hbm, a_v, sem_a)
        cp_b = pltpu.async_copy(b_hbm, b_v, sem_b)
        cp_t = pltpu.async_copy(atoms_hbm.at[pl.ds(batch * _N, _N)], at_v,
                                sem_t)
        cp_t.wait()
        cp_a.wait()
        cp_b.wait()

        ajs = [at_v[pl.ds(jc * 16, 16)] for jc in range(_N // 16)]

        def row_body(r):
            i_splat = jnp.full((16,), i0, jnp.int32) + r
            ai = plsc.load_gather(at_v, [i_splat])
            for jc, aj in enumerate(ajs):
                ag_v[r, pl.ds(jc * 16, 16)] = plsc.load_gather(a_v, [ai, aj])
                bg_v[r, pl.ds(jc * 16, 16)] = plsc.load_gather(b_v, [ai, aj])

        plsc.parallel_loop(0, _RPW, 1, unroll=4)(row_body)
        out_a = pltpu.async_copy(ag_v, ag_hbm.at[pl.ds(row0, _RPW)], sem_a)
        out_b = pltpu.async_copy(bg_v, bg_hbm.at[pl.ds(row0, _RPW)], sem_b)
        out_a.wait()
        out_b.wait()

    return k(atoms_flat, a_pad, b_pad)


_LOG2E = 1.4426950408889634


def _tc_rbf(ag2, bg2, d2, mu2, sigma2):
    rows_per_block = 128
    grid = (_ROWS // rows_per_block,)

    def body(ag_ref, bg_ref, d_ref, mu_ref, sig_ref, o_ref, const_ref):
        @pl.when(pl.program_id(0) == 0)
        def _():
            sig = sig_ref[...]
            const_ref[0:1, :] = mu_ref[...]
            const_ref[1:2, :] = (-0.5 * _LOG2E) / (sig * sig)
            const_ref[2:3, :] = -jnp.log2((jnp.abs(sig) + EPS) * sqrt(2.0 * pi))

        mu = const_ref[0:1, :].reshape(1, 1, NKERNEL)
        s2 = const_ref[1:2, :].reshape(1, 1, NKERNEL)
        lc = const_ref[2:3, :].reshape(1, 1, NKERNEL)
        d = d_ref[...]
        d = jnp.where(jnp.isnan(d), jnp.float32(0.0), d)
        d = jnp.where(d == jnp.inf, jnp.float32(POSINF), d)
        d = jnp.where(d == -jnp.inf, _F32_MIN, d)
        x = ag_ref[...] * d + bg_ref[...]
        u = x[:, :, None] - mu
        o_ref[...] = jnp.exp2((u * u) * s2 + lc)

    return pl.pallas_call(
        body,
        grid=grid,
        in_specs=[
            pl.BlockSpec((rows_per_block, _N), lambda i: (i, 0)),
            pl.BlockSpec((rows_per_block, _N), lambda i: (i, 0)),
            pl.BlockSpec((rows_per_block, _N), lambda i: (i, 0)),
            pl.BlockSpec((1, NKERNEL), lambda i: (0, 0)),
            pl.BlockSpec((1, NKERNEL), lambda i: (0, 0)),
        ],
        out_specs=pl.BlockSpec((rows_per_block, _N, NKERNEL),
                               lambda i: (i, 0, 0)),
        out_shape=jax.ShapeDtypeStruct((_ROWS, _N, NKERNEL), jnp.float32),
        scratch_shapes=[pltpu.VMEM((8, NKERNEL), jnp.float32)],
    )(ag2, bg2, d2, mu2, sigma2)


@jax.jit
def kernel(atoms, distances, mu, sigma, a, b):
    atoms_flat = atoms.reshape(-1).astype(jnp.int32)
    a_pad = jnp.pad(a, ((0, 0), (0, _NAPAD - _NA)))
    b_pad = jnp.pad(b, ((0, 0), (0, _NAPAD - _NA)))
    d2 = distances.reshape(_ROWS, _N)
    ag2, bg2 = _sc_gather(atoms_flat, a_pad, b_pad)
    g = _tc_rbf(ag2, bg2, d2, mu.reshape(1, NKERNEL), sigma.reshape(1, NKERNEL))
    return g.reshape(_B, _N, _N, NKERNEL)
